# Initial kernel scaffold; baseline (speedup 1.0000x reference)
import jax
import jax.numpy as jnp
from jax.experimental import pallas as pl


def kernel(z, embeddings):
    raise NotImplementedError("write your pallas kernel here")


# speedup vs baseline: 9.0792x; 9.0792x over previous
import functools

import jax
import jax.numpy as jnp
from jax import lax
from jax.experimental import pallas as pl
from jax.experimental.pallas import tpu as pltpu
from jax.experimental.pallas import tpu_sc as plsc

_NE = 8192
_ED = 256
_N = 8192
_CC = 0.25

_N_BLK = 512
_K_BLK = 2048
_N_T = _N // _N_BLK
_K_T = _NE // _K_BLK


def _argmin_body(z_ref, et_ref, z2_ref, e2_ref, idx_ref, loss_ref,
                 runmin, runidx):
    i = pl.program_id(0)
    j = pl.program_id(1)
    m = lax.dot_general(z_ref[...].astype(jnp.bfloat16),
                        et_ref[...].astype(jnp.bfloat16),
                        (((1,), (0,)), ((), ())),
                        preferred_element_type=jnp.float32)
    d = (z2_ref[...] - 2.0 * m) + e2_ref[...]
    tmin = jnp.min(d, axis=1, keepdims=True)
    lane = lax.broadcasted_iota(jnp.int32, (_N_BLK, _K_BLK), 1)
    tidx = jnp.min(jnp.where(d == tmin, lane, _NE), axis=1,
                   keepdims=True) + j * _K_BLK

    @pl.when(j == 0)
    def _():
        runmin[...] = tmin
        runidx[...] = tidx

    @pl.when(j > 0)
    def _():
        better = tmin < runmin[...]
        runidx[...] = jnp.where(better, tidx, runidx[...])
        runmin[...] = jnp.where(better, tmin, runmin[...])

    @pl.when(j == _K_T - 1)
    def _():
        idx_ref[...] = runidx[...]
        part = jnp.sum(runmin[...])
        prev = jnp.where(i == 0, jnp.float32(0.0), loss_ref[0, 0])
        tot = prev + part
        scale = jnp.float32((1.0 + _CC) / (_N * _ED))
        loss_ref[0, 0] = jnp.where(i == _N_T - 1, tot * scale, tot)


def _argmin_call(zf, embT, z2, e2):
    return pl.pallas_call(
        _argmin_body,
        grid=(_N_T, _K_T),
        in_specs=[
            pl.BlockSpec((_N_BLK, _ED), lambda i, j: (i, 0)),
            pl.BlockSpec((_ED, _K_BLK), lambda i, j: (0, j)),
            pl.BlockSpec((_N_BLK, 1), lambda i, j: (i, 0)),
            pl.BlockSpec((1, _K_BLK), lambda i, j: (0, j)),
        ],
        out_specs=[
            pl.BlockSpec((_N_BLK, 1), lambda i, j: (i, 0)),
            pl.BlockSpec(memory_space=pltpu.SMEM),
        ],
        out_shape=[
            jax.ShapeDtypeStruct((_N, 1), jnp.int32),
            jax.ShapeDtypeStruct((1, 1), jnp.float32),
        ],
        scratch_shapes=[
            pltpu.VMEM((_N_BLK, 1), jnp.float32),
            pltpu.VMEM((_N_BLK, 1), jnp.int32),
        ],
        compiler_params=pltpu.CompilerParams(
            dimension_semantics=("arbitrary", "arbitrary")),
    )(zf, embT, z2, e2)


_SC_CHUNK = 128


def _sc_gather_call(emb, idx):
    mesh = plsc.VectorSubcoreMesh(core_axis_name="c", subcore_axis_name="s")
    info = plsc.get_sparse_core_info()
    nw = info.num_cores * info.num_subcores
    per_w = _N // nw
    n_ch = per_w // _SC_CHUNK

    @functools.partial(
        pl.kernel, mesh=mesh,
        out_type=jax.ShapeDtypeStruct((_N, _ED), jnp.float32),
        scratch_types=[
            pltpu.VMEM((_SC_CHUNK,), jnp.int32),
            pltpu.VMEM((_SC_CHUNK, _ED), jnp.float32),
            pltpu.SemaphoreType.DMA,
        ],
    )
    def gather(emb_hbm, idx_hbm, out_hbm, idx_v, rows_v, sem):
        wid = lax.axis_index("s") * info.num_cores + lax.axis_index("c")
        base = wid * per_w
        for ch in range(n_ch):
            off = base + ch * _SC_CHUNK
            pltpu.sync_copy(idx_hbm.at[pl.ds(off, _SC_CHUNK)], idx_v)
            pltpu.async_copy(emb_hbm.at[idx_v], rows_v, sem).wait()
            pltpu.sync_copy(rows_v, out_hbm.at[pl.ds(off, _SC_CHUNK)])

    return gather(emb, idx)


def kernel(z, embeddings):
    z_perm = jnp.transpose(z, (0, 2, 3, 4, 1))
    flat_z = z_perm.reshape(_N, _ED)
    z2 = jnp.sum(flat_z ** 2, axis=1, keepdims=True)
    e2 = jnp.sum(embeddings ** 2, axis=1)[None, :]
    embT = embeddings.T
    idx2d, loss = _argmin_call(flat_z, embT, z2, e2)
    idx = idx2d.reshape(_N)
    quantized = _sc_gather_call(embeddings, idx)
    quantized_out = jnp.transpose(
        quantized.reshape(2, 4, 32, 32, _ED), (0, 4, 1, 2, 3))
    return quantized_out, loss.reshape(()), idx
